# Initial kernel scaffold; baseline (speedup 1.0000x reference)
import jax
import jax.numpy as jnp
from jax.experimental import pallas as pl


def kernel(nucleotides, region_mask, emb_table, region_weights):
    raise NotImplementedError("write your pallas kernel here")


# speedup vs baseline: 7.8987x; 7.8987x over previous
import functools

import jax
import jax.numpy as jnp
from jax import lax
from jax.experimental import pallas as pl
from jax.experimental.pallas import tpu as pltpu
from jax.experimental.pallas import tpu_sc as plsc

D = 32
B = 1024
S = 2048
SP = S - 1
NC, NS = 2, 16
NW = NC * NS
ROWS_PER_W = B // NW


def _fuse_body(emb_ref, rw_ref, out_ref):
    k = lax.broadcasted_iota(jnp.int32, (48, 16), 0)
    d16 = lax.broadcasted_iota(jnp.int32, (48, 16), 1)
    oh_d = (k // 3 == d16).astype(jnp.float32)
    k3 = lax.broadcasted_iota(jnp.int32, (48, 3), 0)
    r3 = lax.broadcasted_iota(jnp.int32, (48, 3), 1)
    oh_r = (k3 % 3 == r3).astype(jnp.float32)
    emb16 = emb_ref[0:16, :]
    ed = jnp.dot(oh_d, emb16, preferred_element_type=jnp.float32)
    wr = jnp.dot(oh_r, rw_ref[...], preferred_element_type=jnp.float32)
    out_ref[...] = ed * wr


def _build_fused(emb_table, region_weights):
    return pl.pallas_call(
        _fuse_body,
        out_shape=jax.ShapeDtypeStruct((48, D), jnp.float32),
    )(emb_table, region_weights)


_MESH = plsc.VectorSubcoreMesh(
    core_axis_name="c", subcore_axis_name="s", num_cores=NC, num_subcores=NS
)


@functools.partial(
    pl.kernel,
    out_type=jax.ShapeDtypeStruct((B, SP, D), jnp.float32),
    mesh=_MESH,
    scratch_types=[
        pltpu.VMEM((S + 16,), jnp.int32),
        pltpu.VMEM((S,), jnp.int32),
        pltpu.VMEM((S,), jnp.int32),
        pltpu.VMEM((S, D), jnp.float32),
        pltpu.SemaphoreType.DMA,
    ],
    compiler_params=pltpu.CompilerParams(use_tc_tiling_on_sc=False),
)
def _sc_lookup(nuc_hbm, rm_hbm, fused_hbm, out_hbm, nuc_v, rm_v, idx_v, rows_v, sem):
    wid = lax.axis_index("s") * NC + lax.axis_index("c")

    def row_body(i, carry):
        b = wid * ROWS_PER_W + i
        pltpu.sync_copy(nuc_hbm.at[b], nuc_v.at[pl.ds(0, S)])
        pltpu.sync_copy(rm_hbm.at[b], rm_v)

        def chunk(j, c2):
            first = nuc_v[pl.ds(j * 16, 16)]
            second = nuc_v[pl.ds(j * 16 + 1, 16)]
            r = rm_v[pl.ds(j * 16, 16)]
            k = first * 12 + second * 3 + r
            k = jnp.minimum(jnp.maximum(k, 0), 47)
            idx_v[pl.ds(j * 16, 16)] = k
            return c2

        lax.fori_loop(0, S // 16, chunk, 0)
        pltpu.async_copy(fused_hbm.at[idx_v], rows_v, sem).wait()
        pltpu.sync_copy(rows_v.at[pl.ds(0, SP)], out_hbm.at[b])
        return carry

    lax.fori_loop(0, ROWS_PER_W, row_body, 0)


def kernel(nucleotides, region_mask, emb_table, region_weights):
    nuc = nucleotides.astype(jnp.int32)
    rm = region_mask.astype(jnp.int32)
    fused = _build_fused(emb_table, region_weights)
    return _sc_lookup(nuc, rm, fused)


# speedup vs baseline: 7.9100x; 1.0014x over previous
import functools

import jax
import jax.numpy as jnp
from jax import lax
from jax.experimental import pallas as pl
from jax.experimental.pallas import tpu as pltpu
from jax.experimental.pallas import tpu_sc as plsc

D = 32
B = 1024
S = 2048
SP = S - 1
NC, NS = 2, 16
NW = NC * NS
ROWS_PER_W = B // NW


def _fuse_body(emb_ref, rw_ref, out_ref):
    k = lax.broadcasted_iota(jnp.int32, (48, 16), 0)
    d16 = lax.broadcasted_iota(jnp.int32, (48, 16), 1)
    oh_d = (k // 3 == d16).astype(jnp.float32)
    k3 = lax.broadcasted_iota(jnp.int32, (48, 3), 0)
    r3 = lax.broadcasted_iota(jnp.int32, (48, 3), 1)
    oh_r = (k3 % 3 == r3).astype(jnp.float32)
    emb16 = emb_ref[0:16, :]
    ed = jnp.dot(oh_d, emb16, preferred_element_type=jnp.float32)
    wr = jnp.dot(oh_r, rw_ref[...], preferred_element_type=jnp.float32)
    out_ref[...] = ed * wr


def _build_fused(emb_table, region_weights):
    return pl.pallas_call(
        _fuse_body,
        out_shape=jax.ShapeDtypeStruct((48, D), jnp.float32),
    )(emb_table, region_weights)


_MESH = plsc.VectorSubcoreMesh(
    core_axis_name="c", subcore_axis_name="s", num_cores=NC, num_subcores=NS
)


@functools.partial(
    pl.kernel,
    out_type=jax.ShapeDtypeStruct((B, SP, D), jnp.float32),
    mesh=_MESH,
    scratch_types=[
        pltpu.VMEM((S + 16,), jnp.int32),
        pltpu.VMEM((S,), jnp.int32),
        pltpu.VMEM((16, 128), jnp.int32),
        pltpu.VMEM((S, D), jnp.float32),
        pltpu.SemaphoreType.DMA,
    ],
    compiler_params=pltpu.CompilerParams(use_tc_tiling_on_sc=False),
)
def _sc_lookup(nuc_hbm, rm_hbm, fused_hbm, out_hbm, nuc_v, rm_v, idx_v, rows_v, sem):
    wid = lax.axis_index("s") * NC + lax.axis_index("c")

    def row_body(i, carry):
        b = wid * ROWS_PER_W + i
        pltpu.sync_copy(nuc_hbm.at[b], nuc_v.at[pl.ds(0, S)])
        pltpu.sync_copy(rm_hbm.at[b], rm_v)

        for j in range(S // 16):
            first = nuc_v[pl.ds(j * 16, 16)]
            second = nuc_v[pl.ds(j * 16 + 1, 16)]
            r = rm_v[pl.ds(j * 16, 16)]
            k = first * 12 + second * 3 + r
            if j == S // 16 - 1:
                k = jnp.minimum(jnp.maximum(k, 0), 47)
            idx_v[j // 8, pl.ds((j % 8) * 16, 16)] = k

        copies = [
            pltpu.async_copy(
                fused_hbm.at[idx_v.at[c]], rows_v.at[pl.ds(c * 128, 128)], sem
            )
            for c in range(16)
        ]
        for cp in copies:
            cp.wait()
        pltpu.sync_copy(rows_v.at[pl.ds(0, SP)], out_hbm.at[b])
        return carry

    lax.fori_loop(0, ROWS_PER_W, row_body, 0)


def kernel(nucleotides, region_mask, emb_table, region_weights):
    nuc = nucleotides.astype(jnp.int32)
    rm = region_mask.astype(jnp.int32)
    fused = _build_fused(emb_table, region_weights)
    return _sc_lookup(nuc, rm, fused)


# speedup vs baseline: 14.8246x; 1.8742x over previous
import functools

import jax
import jax.numpy as jnp
from jax import lax
from jax.experimental import pallas as pl
from jax.experimental.pallas import tpu as pltpu
from jax.experimental.pallas import tpu_sc as plsc

D = 32
B = 1024
S = 2048
SP = S - 1
NC, NS = 2, 16
NW = NC * NS
ROWS_PER_W = B // NW


def _fuse_body(emb_ref, rw_ref, out_ref):
    k = lax.broadcasted_iota(jnp.int32, (48, 16), 0)
    d16 = lax.broadcasted_iota(jnp.int32, (48, 16), 1)
    oh_d = (k // 3 == d16).astype(jnp.float32)
    k3 = lax.broadcasted_iota(jnp.int32, (48, 3), 0)
    r3 = lax.broadcasted_iota(jnp.int32, (48, 3), 1)
    oh_r = (k3 % 3 == r3).astype(jnp.float32)
    emb16 = emb_ref[0:16, :]
    ed = jnp.dot(oh_d, emb16, preferred_element_type=jnp.float32)
    wr = jnp.dot(oh_r, rw_ref[...], preferred_element_type=jnp.float32)
    f48 = ed * wr
    kr = lax.broadcasted_iota(jnp.int32, (NW * 48, 48), 0)
    c48 = lax.broadcasted_iota(jnp.int32, (NW * 48, 48), 1)
    oh_rep = (kr % 48 == c48).astype(jnp.float32)
    out_ref[...] = jnp.dot(oh_rep, f48, preferred_element_type=jnp.float32)


def _build_fused(emb_table, region_weights):
    return pl.pallas_call(
        _fuse_body,
        out_shape=jax.ShapeDtypeStruct((NW * 48, D), jnp.float32),
    )(emb_table, region_weights)


_MESH = plsc.VectorSubcoreMesh(
    core_axis_name="c", subcore_axis_name="s", num_cores=NC, num_subcores=NS
)


@functools.partial(
    pl.kernel,
    out_type=jax.ShapeDtypeStruct((B, SP, D), jnp.float32),
    mesh=_MESH,
    scratch_types=[
        pltpu.VMEM((S + 16,), jnp.int32),
        pltpu.VMEM((S,), jnp.int32),
        pltpu.VMEM((16, 128), jnp.int32),
        pltpu.VMEM((S, D), jnp.float32),
        pltpu.SemaphoreType.DMA,
    ],
    compiler_params=pltpu.CompilerParams(use_tc_tiling_on_sc=False),
)
def _sc_lookup(nuc_hbm, rm_hbm, fused_hbm, out_hbm, nuc_v, rm_v, idx_v, rows_v, sem):
    wid = lax.axis_index("s") * NC + lax.axis_index("c")

    tbl_base = wid * 48

    def row_body(i, carry):
        b = wid * ROWS_PER_W + i
        pltpu.sync_copy(nuc_hbm.at[b], nuc_v.at[pl.ds(0, S)])
        pltpu.sync_copy(rm_hbm.at[b], rm_v)

        for j in range(S // 16):
            first = nuc_v[pl.ds(j * 16, 16)]
            second = nuc_v[pl.ds(j * 16 + 1, 16)]
            r = rm_v[pl.ds(j * 16, 16)]
            k = first * 12 + second * 3 + r
            if j == S // 16 - 1:
                k = jnp.minimum(jnp.maximum(k, 0), 47)
            idx_v[j // 8, pl.ds((j % 8) * 16, 16)] = k + tbl_base

        copies = [
            pltpu.async_copy(
                fused_hbm.at[idx_v.at[c]], rows_v.at[pl.ds(c * 128, 128)], sem
            )
            for c in range(16)
        ]
        for cp in copies:
            cp.wait()
        pltpu.sync_copy(rows_v.at[pl.ds(0, SP)], out_hbm.at[b])
        return carry

    lax.fori_loop(0, ROWS_PER_W, row_body, 0)


def kernel(nucleotides, region_mask, emb_table, region_weights):
    nuc = nucleotides.astype(jnp.int32)
    rm = region_mask.astype(jnp.int32)
    fused = _build_fused(emb_table, region_weights)
    return _sc_lookup(nuc, rm, fused)


# speedup vs baseline: 15.7984x; 1.0657x over previous
import functools

import jax
import jax.numpy as jnp
from jax import lax
from jax.experimental import pallas as pl
from jax.experimental.pallas import tpu as pltpu
from jax.experimental.pallas import tpu_sc as plsc

D = 32
B = 1024
S = 2048
SP = S - 1
NC, NS = 2, 16
NW = NC * NS
ROWS_PER_W = B // NW


def _fuse_body(emb_ref, rw_ref, out_ref):
    k = lax.broadcasted_iota(jnp.int32, (48, 16), 0)
    d16 = lax.broadcasted_iota(jnp.int32, (48, 16), 1)
    oh_d = (k // 3 == d16).astype(jnp.float32)
    k3 = lax.broadcasted_iota(jnp.int32, (48, 3), 0)
    r3 = lax.broadcasted_iota(jnp.int32, (48, 3), 1)
    oh_r = (k3 % 3 == r3).astype(jnp.float32)
    emb16 = emb_ref[0:16, :]
    ed = jnp.dot(oh_d, emb16, preferred_element_type=jnp.float32)
    wr = jnp.dot(oh_r, rw_ref[...], preferred_element_type=jnp.float32)
    f48 = ed * wr
    kr = lax.broadcasted_iota(jnp.int32, (NW * 48, 48), 0)
    c48 = lax.broadcasted_iota(jnp.int32, (NW * 48, 48), 1)
    oh_rep = (kr % 48 == c48).astype(jnp.float32)
    out_ref[...] = jnp.dot(oh_rep, f48, preferred_element_type=jnp.float32)


def _build_fused(emb_table, region_weights):
    return pl.pallas_call(
        _fuse_body,
        out_shape=jax.ShapeDtypeStruct((NW * 48, D), jnp.float32),
    )(emb_table, region_weights)


_MESH = plsc.VectorSubcoreMesh(
    core_axis_name="c", subcore_axis_name="s", num_cores=NC, num_subcores=NS
)


@functools.partial(
    pl.kernel,
    out_type=jax.ShapeDtypeStruct((B, SP, D), jnp.float32),
    mesh=_MESH,
    scratch_types=[
        pltpu.VMEM((S + 16,), jnp.int32),
        pltpu.VMEM((S,), jnp.int32),
        pltpu.VMEM((16, 128), jnp.int32),
        pltpu.VMEM((S, D), jnp.float32),
        pltpu.SemaphoreType.DMA,
    ],
    compiler_params=pltpu.CompilerParams(use_tc_tiling_on_sc=False),
)
def _sc_lookup(nuc_hbm, rm_hbm, fused_hbm, out_hbm, nuc_v, rm_v, idx_v, rows_v, sem):
    wid = lax.axis_index("s") * NC + lax.axis_index("c")

    tbl_base = wid * 48

    def row_body(i, carry):
        b = wid * ROWS_PER_W + i
        pltpu.sync_copy(nuc_hbm.at[b], nuc_v.at[pl.ds(0, S)])
        pltpu.sync_copy(rm_hbm.at[b], rm_v)

        for j in range(S // 16):
            first = nuc_v[pl.ds(j * 16, 16)]
            second = nuc_v[pl.ds(j * 16 + 1, 16)]
            r = rm_v[pl.ds(j * 16, 16)]
            k = first * 12 + second * 3 + r
            if j == S // 16 - 1:
                k = jnp.minimum(jnp.maximum(k, 0), 47)
            idx_v[j // 8, pl.ds((j % 8) * 16, 16)] = k + (tbl_base + (j % NW) * 48) % (NW * 48)

        copies = [
            pltpu.async_copy(
                fused_hbm.at[idx_v.at[c]], rows_v.at[pl.ds(c * 128, 128)], sem
            )
            for c in range(16)
        ]
        for cp in copies:
            cp.wait()
        pltpu.sync_copy(rows_v.at[pl.ds(0, SP)], out_hbm.at[b])
        return carry

    lax.fori_loop(0, ROWS_PER_W, row_body, 0)


def kernel(nucleotides, region_mask, emb_table, region_weights):
    nuc = nucleotides.astype(jnp.int32)
    rm = region_mask.astype(jnp.int32)
    fused = _build_fused(emb_table, region_weights)
    return _sc_lookup(nuc, rm, fused)


# speedup vs baseline: 18.2295x; 1.1539x over previous
import functools

import jax
import jax.numpy as jnp
from jax import lax
from jax.experimental import pallas as pl
from jax.experimental.pallas import tpu as pltpu
from jax.experimental.pallas import tpu_sc as plsc

D = 32
B = 1024
S = 2048
SP = S - 1
NC, NS = 2, 16
NW = NC * NS
ROWS_PER_W = B // NW


def _fuse_body(emb_ref, rw_ref, out_ref):
    k = lax.broadcasted_iota(jnp.int32, (48, 16), 0)
    d16 = lax.broadcasted_iota(jnp.int32, (48, 16), 1)
    oh_d = (k // 3 == d16).astype(jnp.float32)
    k3 = lax.broadcasted_iota(jnp.int32, (48, 3), 0)
    r3 = lax.broadcasted_iota(jnp.int32, (48, 3), 1)
    oh_r = (k3 % 3 == r3).astype(jnp.float32)
    emb16 = emb_ref[0:16, :]
    ed = jnp.dot(oh_d, emb16, preferred_element_type=jnp.float32)
    wr = jnp.dot(oh_r, rw_ref[...], preferred_element_type=jnp.float32)
    f48 = ed * wr
    kr = lax.broadcasted_iota(jnp.int32, (NW * 48, 48), 0)
    c48 = lax.broadcasted_iota(jnp.int32, (NW * 48, 48), 1)
    oh_rep = (kr % 48 == c48).astype(jnp.float32)
    out_ref[...] = jnp.dot(oh_rep, f48, preferred_element_type=jnp.float32)


def _build_fused(emb_table, region_weights):
    return pl.pallas_call(
        _fuse_body,
        out_shape=jax.ShapeDtypeStruct((NW * 48, D), jnp.float32),
    )(emb_table, region_weights)


_MESH = plsc.VectorSubcoreMesh(
    core_axis_name="c", subcore_axis_name="s", num_cores=NC, num_subcores=NS
)


@functools.partial(
    pl.kernel,
    out_type=jax.ShapeDtypeStruct((B * S, D), jnp.float32),
    mesh=_MESH,
    scratch_types=[
        pltpu.VMEM((S + 16,), jnp.int32),
        pltpu.VMEM((S,), jnp.int32),
        pltpu.VMEM((16, 128), jnp.int32),
        pltpu.VMEM((S, D), jnp.float32),
        pltpu.SemaphoreType.DMA,
    ],
    compiler_params=pltpu.CompilerParams(use_tc_tiling_on_sc=False),
)
def _sc_lookup(nuc_hbm, rm_hbm, fused_hbm, out_hbm, nuc_v, rm_v, idx_v, rows_v, sem):
    wid = lax.axis_index("s") * NC + lax.axis_index("c")

    tbl_base = wid * 48

    def row_body(i, carry):
        b = wid * ROWS_PER_W + i
        pltpu.sync_copy(nuc_hbm.at[b], nuc_v.at[pl.ds(0, S)])
        pltpu.sync_copy(rm_hbm.at[b], rm_v)

        for j in range(S // 16):
            first = nuc_v[pl.ds(j * 16, 16)]
            second = nuc_v[pl.ds(j * 16 + 1, 16)]
            r = rm_v[pl.ds(j * 16, 16)]
            k = first * 12 + second * 3 + r
            if j == S // 16 - 1:
                k = jnp.minimum(jnp.maximum(k, 0), 47)
            idx_v[j // 8, pl.ds((j % 8) * 16, 16)] = k + (tbl_base + (j % NW) * 48) % (NW * 48)

        copies = [
            pltpu.async_copy(
                fused_hbm.at[idx_v.at[c]], rows_v.at[pl.ds(c * 128, 128)], sem
            )
            for c in range(16)
        ]
        for cp in copies:
            cp.wait()
        pltpu.sync_copy(rows_v.at[pl.ds(0, SP)], out_hbm.at[pl.ds(b * S, SP)])
        return carry

    lax.fori_loop(0, ROWS_PER_W, row_body, 0)


def kernel(nucleotides, region_mask, emb_table, region_weights):
    nuc = nucleotides.astype(jnp.int32)
    rm = region_mask.astype(jnp.int32)
    fused = _build_fused(emb_table, region_weights)
    flat = _sc_lookup(nuc, rm, fused)
    return flat.reshape(B, S, D)[:, :SP, :]
